# baseline (speedup 1.0000x reference)
import jax
import jax.numpy as jnp
from jax.experimental import pallas as pl
from jax.experimental.pallas import tpu as pltpu

_N = 20000
_ROWS, _LANES = 160, 128
_NPAD = _ROWS * _LANES
_NCLS = 80
_K = 1000
_SCORE_THRESH = 0.2
_NMS_THRESH = 0.6
_NUM_OUT = 100
_IMG = 1024.0
_PIB = "promise_in_bounds"


def _fcos_body(lg_ref, aux_ref, out_ref, g_ref, s0_r, t0_r, cnt_r):
    row = jax.lax.broadcasted_iota(jnp.int32, (_ROWS, _LANES), 0)
    col = jax.lax.broadcasted_iota(jnp.int32, (_ROWS, _LANES), 1)
    gidx = row * _LANES + col

    maxl = lg_ref[0]
    lab = jnp.zeros((_ROWS, _LANES), jnp.int32)
    for c in range(1, _NCLS):
        x = lg_ref[c]
        upd = x > maxl
        maxl = jnp.where(upd, x, maxl)
        lab = jnp.where(upd, c, lab)
    labf = lab.astype(jnp.float32)
    ctr = aux_ref[0]
    s = jnp.sqrt(jax.nn.sigmoid(maxl) * jax.nn.sigmoid(ctr))
    s = jnp.where(s > _SCORE_THRESH, s, 0.0)
    s = jnp.where(gidx < _N, s, 0.0)

    cx = (aux_ref[5] + aux_ref[7]) / 2.0
    cy = (aux_ref[6] + aux_ref[8]) / 2.0
    bx1 = jnp.clip(cx - aux_ref[1], 0.0, _IMG)
    by1 = jnp.clip(cy - aux_ref[2], 0.0, _IMG)
    bx2 = jnp.clip(cx + aux_ref[3], 0.0, _IMG)
    by2 = jnp.clip(cy + aux_ref[4], 0.0, _IMG)

    si = jax.lax.bitcast_convert_type(s, jnp.int32)

    def count(mask):
        return jnp.sum(mask.astype(jnp.int32))

    t = jnp.int32(0)
    for b in range(30, -1, -1):
        tc = t | jnp.int32(1 << b)
        t = jnp.where(count(si >= tc) >= _K, tc, t)
    eqt = si == t
    need = _K - count(si > t)
    x_cut = jnp.int32(0)
    for b in range(14, -1, -1):
        xc = x_cut | jnp.int32(1 << b)
        x_cut = jnp.where(count(eqt & (gidx < xc)) < need, xc, x_cut)
    cand = (si > t) | (eqt & (gidx <= x_cut))

    m4 = jnp.maximum(jnp.maximum(bx1, by1), jnp.maximum(bx2, by2))
    mc = jnp.max(jnp.where(cand, m4, 0.0))

    inc = cand.astype(jnp.int32)
    for k in (1, 2, 4, 8, 16, 32, 64):
        inc = inc + jnp.concatenate(
            [jnp.zeros((_ROWS, k), jnp.int32), inc[:, :_LANES - k]], axis=1)
    rowcnt = inc[:, _LANES - 1:_LANES]
    rr = rowcnt
    for k in (1, 2, 4, 8, 16, 32, 64, 128):
        rr = rr + jnp.concatenate(
            [jnp.zeros((k, 1), jnp.int32), rr[:_ROWS - k, :]], axis=0)
    ro = rr - rowcnt
    s0_r[...] = jnp.bitwise_and(ro, _LANES - 1)
    t0_r[...] = ro // _LANES
    cnt_r[...] = rowcnt
    qp1 = col + 1
    sr = jnp.zeros((_ROWS, _LANES), jnp.int32)
    for b in (128, 64, 32, 16, 8, 4, 2, 1):
        tt = jnp.minimum(sr + (b - 1), _LANES - 1)
        v = jnp.take_along_axis(inc, tt, axis=1, mode=_PIB)
        sr = jnp.where((v < qp1) & (sr + b <= _LANES), sr + b, sr)
    srclane = jnp.minimum(sr, _LANES - 1)

    def cg(x):
        return jnp.take_along_axis(x, srclane, axis=1, mode=_PIB)

    lg_ref[0] = cg(bx1)
    lg_ref[1] = cg(by1)
    lg_ref[2] = cg(bx2)
    lg_ref[3] = cg(by2)
    lg_ref[4] = cg(jnp.where(cand, s, 0.0))
    lg_ref[5] = cg(labf)

    g_ref[...] = jnp.zeros((6, 16, _LANES), jnp.float32)
    lanev = jax.lax.broadcasted_iota(jnp.int32, (1, _LANES), 1)

    def put(r, _):
        s0 = jnp.sum(s0_r[pl.ds(r, 1), :])
        t0 = jnp.sum(t0_r[pl.ds(r, 1), :])
        cn = jnp.sum(cnt_r[pl.ds(r, 1), :])
        qs = lanev - s0
        qs = jnp.where(qs < 0, qs + _LANES, qs)
        vm = qs < cn
        m_hi = vm & (lanev >= s0)
        m_lo = vm & (lanev < s0)
        for f in range(6):
            seg = lg_ref[f, pl.ds(r, 1), :]
            rot = pltpu.roll(seg, s0, axis=1)
            cur0 = g_ref[f, pl.ds(t0, 1), :]
            g_ref[f, pl.ds(t0, 1), :] = jnp.where(m_hi, rot, cur0)
            cur1 = g_ref[f, pl.ds(t0 + 1, 1), :]
            g_ref[f, pl.ds(t0 + 1, 1), :] = jnp.where(m_lo, rot, cur1)
        return 0

    jax.lax.fori_loop(0, _ROWS, put, 0)

    row8 = jax.lax.broadcasted_iota(jnp.int32, (8, _LANES), 0)
    col8 = jax.lax.broadcasted_iota(jnp.int32, (8, _LANES), 1)
    slot = row8 * _LANES + col8
    gbx1 = g_ref[0, 0:8, :]
    gby1 = g_ref[1, 0:8, :]
    gbx2 = g_ref[2, 0:8, :]
    gby2 = g_ref[3, 0:8, :]
    gsc = g_ref[4, 0:8, :]
    glab = g_ref[5, 0:8, :]
    goff = glab * (mc + 1.0)
    gnx1 = gbx1 + goff
    gny1 = gby1 + goff
    gnx2 = gbx2 + goff
    gny2 = gby2 + goff
    garea = (gnx2 - gnx1) * (gny2 - gny1)
    live0 = jnp.where(slot < _K, gsc, -jnp.inf)
    lanei = jax.lax.broadcasted_iota(jnp.int32, (1, _LANES), 1)

    def step(i, carry):
        live, first = carry
        m = jnp.max(live)
        j = jnp.min(jnp.where(live == m, slot, _NPAD))
        j = jnp.where(m == -jnp.inf, first, j)
        first = jnp.where(i == 0, j, first)
        jr = j // _LANES
        jc = j - jr * _LANES
        jr_f = jnp.full((8, _LANES), jr, jnp.int32)
        jc_f = jnp.full((8, _LANES), jc, jnp.int32)

        def pick(x):
            g = jnp.take_along_axis(x, jr_f, axis=0, mode=_PIB)
            return jnp.take_along_axis(g, jc_f, axis=1, mode=_PIB)

        px1b = pick(gbx1)
        py1b = pick(gby1)
        px2b = pick(gbx2)
        py2b = pick(gby2)
        psc = pick(gsc)
        plab = pick(glab)
        poff = plab * (mc + 1.0)
        px1 = px1b + poff
        py1 = py1b + poff
        px2 = px2b + poff
        py2 = py2b + poff
        parea = (px2 - px1) * (py2 - py1)
        ltx = jnp.maximum(px1, gnx1)
        lty = jnp.maximum(py1, gny1)
        rbx = jnp.minimum(px2, gnx2)
        rby = jnp.minimum(py2, gny2)
        w = jnp.maximum(rbx - ltx, 0.0)
        h = jnp.maximum(rby - lty, 0.0)
        inter = w * h
        iou = inter / (parea + garea - inter + 1e-9)
        live = jnp.where((iou > _NMS_THRESH) | (slot == j), -jnp.inf, live)
        rowout = jnp.where(lanei == 0, px1b[0:1, :],
                 jnp.where(lanei == 1, py1b[0:1, :],
                 jnp.where(lanei == 2, px2b[0:1, :],
                 jnp.where(lanei == 3, py2b[0:1, :],
                 jnp.where(lanei == 4, psc[0:1, :],
                 jnp.where(lanei == 5, plab[0:1, :], 0.0))))))
        out_ref[pl.ds(i, 1), :] = rowout
        return live, first

    jax.lax.fori_loop(0, _NUM_OUT, step, (live0, jnp.int32(0)))


@jax.jit
def kernel(cls_logits, bbox_regression, bbox_ctrness, anchors):
    def fold(x):
        f = x.shape[1]
        xp = jnp.pad(x, ((0, _NPAD - _N), (0, 0)))
        return xp.T.reshape(f, _ROWS, _LANES)

    lg = fold(cls_logits)
    aux = fold(jnp.concatenate(
        [bbox_ctrness, bbox_regression, anchors], axis=1))

    out = pl.pallas_call(
        _fcos_body,
        in_specs=[pl.BlockSpec(memory_space=pltpu.VMEM)] * 2,
        out_specs=pl.BlockSpec(memory_space=pltpu.VMEM),
        out_shape=jax.ShapeDtypeStruct((104, _LANES), jnp.float32),
        scratch_shapes=[
            pltpu.VMEM((6, 16, _LANES), jnp.float32),
            pltpu.VMEM((_ROWS, 1), jnp.int32),
            pltpu.VMEM((_ROWS, 1), jnp.int32),
            pltpu.VMEM((_ROWS, 1), jnp.int32),
        ],
    )(lg, aux)
    dets = out[:_NUM_OUT, :5]
    labels_out = out[:_NUM_OUT, 5].astype(jnp.int32)
    return dets, labels_out


# speedup vs baseline: 1.0284x; 1.0284x over previous
import jax
import jax.numpy as jnp
from jax.experimental import pallas as pl
from jax.experimental.pallas import tpu as pltpu

_N = 20000
_ROWS, _LANES = 160, 128
_NPAD = _ROWS * _LANES
_NCLS = 80
_K = 1000
_SCORE_THRESH = 0.2
_NMS_THRESH = 0.6
_NUM_OUT = 100
_IMG = 1024.0


def _fcos_body(lg_ref, aux_ref, out_ref, nx1_r, ny1_r, nx2_r, ny2_r, area_r):
    row = jax.lax.broadcasted_iota(jnp.int32, (_ROWS, _LANES), 0)
    col = jax.lax.broadcasted_iota(jnp.int32, (_ROWS, _LANES), 1)
    gidx = col * _ROWS + row

    maxl = lg_ref[0]
    lab = jnp.zeros((_ROWS, _LANES), jnp.int32)
    for c in range(1, _NCLS):
        x = lg_ref[c]
        upd = x > maxl
        maxl = jnp.where(upd, x, maxl)
        lab = jnp.where(upd, c, lab)
    labf = lab.astype(jnp.float32)
    ctr = aux_ref[0]
    s = jnp.sqrt(jax.nn.sigmoid(maxl) * jax.nn.sigmoid(ctr))
    s = jnp.where(s > _SCORE_THRESH, s, 0.0)
    s = jnp.where(gidx < _N, s, 0.0)

    cx = (aux_ref[5] + aux_ref[7]) / 2.0
    cy = (aux_ref[6] + aux_ref[8]) / 2.0
    bx1 = jnp.clip(cx - aux_ref[1], 0.0, _IMG)
    by1 = jnp.clip(cy - aux_ref[2], 0.0, _IMG)
    bx2 = jnp.clip(cx + aux_ref[3], 0.0, _IMG)
    by2 = jnp.clip(cy + aux_ref[4], 0.0, _IMG)

    si = jax.lax.bitcast_convert_type(s, jnp.int32)

    def count(mask):
        return jnp.sum(mask.astype(jnp.int32))

    t = jnp.int32(0)
    for b in range(30, -1, -1):
        tc = t | jnp.int32(1 << b)
        t = jnp.where(count(si >= tc) >= _K, tc, t)
    eqt = si == t
    need = _K - count(si > t)
    x_cut = jnp.int32(0)
    for b in range(14, -1, -1):
        xc = x_cut | jnp.int32(1 << b)
        x_cut = jnp.where(count(eqt & (gidx < xc)) < need, xc, x_cut)
    cand = (si > t) | (eqt & (gidx <= x_cut))

    m4 = jnp.maximum(jnp.maximum(bx1, by1), jnp.maximum(bx2, by2))
    mc = jnp.max(jnp.where(cand, m4, 0.0))
    off = labf * (mc + 1.0)
    nx1 = bx1 + off
    ny1 = by1 + off
    nx2 = bx2 + off
    ny2 = by2 + off
    nx1_r[...] = nx1
    ny1_r[...] = ny1
    nx2_r[...] = nx2
    ny2_r[...] = ny2
    area_r[...] = (nx2 - nx1) * (ny2 - ny1)
    live0 = jnp.where(cand, s, -jnp.inf)
    lg_ref[0] = bx1
    lg_ref[1] = by1
    lg_ref[2] = bx2
    lg_ref[3] = by2
    lg_ref[4] = jnp.where(cand, s, 0.0)
    lg_ref[5] = labf

    lanei = jax.lax.broadcasted_iota(jnp.int32, (1, _LANES), 1)

    def step(i, carry):
        live, first = carry
        colmax = jnp.max(live, axis=0, keepdims=True)
        colarg = jnp.argmax(live, axis=0, keepdims=True)
        m = jnp.max(colmax)
        idx = jnp.min(jnp.where(colmax == m,
                                lanei * _ROWS + colarg.astype(jnp.int32),
                                _NPAD))
        idx = jnp.where(m == -jnp.inf, first, idx)
        first = jnp.where(i == 0, idx, first)
        cf = idx // _ROWS
        rf = idx - cf * _ROWS
        cf_v = jnp.full((8, _LANES), cf, jnp.int32)

        def pick(f):
            rowv = lg_ref[f, pl.ds(rf, 1), :]
            g = jnp.take_along_axis(jnp.broadcast_to(rowv, (8, _LANES)),
                                    cf_v, axis=1,
                                    mode="promise_in_bounds")
            return g[0:1, :]

        px1b = pick(0)
        py1b = pick(1)
        px2b = pick(2)
        py2b = pick(3)
        psc = pick(4)
        plab = pick(5)
        poff = plab * (mc + 1.0)
        px1 = px1b + poff
        py1 = py1b + poff
        px2 = px2b + poff
        py2 = py2b + poff
        parea = (px2 - px1) * (py2 - py1)
        ltx = jnp.maximum(px1, nx1_r[...])
        lty = jnp.maximum(py1, ny1_r[...])
        rbx = jnp.minimum(px2, nx2_r[...])
        rby = jnp.minimum(py2, ny2_r[...])
        w = jnp.maximum(rbx - ltx, 0.0)
        h = jnp.maximum(rby - lty, 0.0)
        inter = w * h
        iou = inter / (parea + area_r[...] - inter + 1e-9)
        live = jnp.where((iou > _NMS_THRESH) | (gidx == idx), -jnp.inf, live)
        rowout = jnp.where(lanei == 0, px1b,
                 jnp.where(lanei == 1, py1b,
                 jnp.where(lanei == 2, px2b,
                 jnp.where(lanei == 3, py2b,
                 jnp.where(lanei == 4, psc,
                 jnp.where(lanei == 5, plab, 0.0))))))
        out_ref[pl.ds(i, 1), :] = rowout
        return live, first

    jax.lax.fori_loop(0, _NUM_OUT, step, (live0, jnp.int32(0)))


@jax.jit
def kernel(cls_logits, bbox_regression, bbox_ctrness, anchors):
    def fold(x):
        f = x.shape[1]
        xp = jnp.pad(x, ((0, _NPAD - _N), (0, 0)))
        return xp.T.reshape(f, _LANES, _ROWS).transpose(0, 2, 1)

    lg = fold(cls_logits)
    aux = fold(jnp.concatenate(
        [bbox_ctrness, bbox_regression, anchors], axis=1))

    out = pl.pallas_call(
        _fcos_body,
        in_specs=[pl.BlockSpec(memory_space=pltpu.VMEM)] * 2,
        out_specs=pl.BlockSpec(memory_space=pltpu.VMEM),
        out_shape=jax.ShapeDtypeStruct((104, _LANES), jnp.float32),
        scratch_shapes=[pltpu.VMEM((_ROWS, _LANES), jnp.float32)] * 5,
    )(lg, aux)
    dets = out[:_NUM_OUT, :5]
    labels_out = out[:_NUM_OUT, 5].astype(jnp.int32)
    return dets, labels_out
